# baseline (speedup 1.0000x reference)
import jax
import jax.numpy as jnp
from jax.experimental import pallas as pl
from jax.experimental.pallas import tpu as pltpu

_B, _D, _E = 8192, 4096, 64
_BLK = 512
_NBLK = _B // _BLK

_NT = (((1,), (1,)), ((), ()))


def _leaky(h):
    return jnp.where(h >= 0, h, 0.01 * h)


def _ntdot(a, w):
    return jax.lax.dot_general(a, w, _NT, preferred_element_type=jnp.float32)


def _gate_kernel(x_ref, w1_ref, b1_ref, w2_ref, b2_ref, w3_ref, b3_ref,
                 w4_ref, b4_ref, out_ref, acc_ref):
    i = pl.program_id(0)
    nsteps = pl.num_programs(0)

    @pl.when(i == 0)
    def _init():
        acc_ref[0] = 0.0

    h = jnp.maximum(_ntdot(x_ref[...], w1_ref[...]) + b1_ref[...], 0.0)
    h = _leaky(_ntdot(h, w2_ref[...]) + b2_ref[...])
    h = _leaky(_ntdot(h, w3_ref[...]) + b3_ref[...])
    logits = _ntdot(h, w4_ref[...]) + b4_ref[...]

    m1 = jnp.max(logits, axis=1, keepdims=True)
    is_max = logits == m1
    dup = jnp.sum(is_max.astype(jnp.float32), axis=1, keepdims=True) > 1.0
    m2_lo = jnp.max(jnp.where(is_max, -jnp.inf, logits), axis=1, keepdims=True)
    m2 = jnp.where(dup, m1, m2_lo)
    acc_ref[0] += jnp.sum(m1) + jnp.sum(m2)

    @pl.when(i < nsteps - 1)
    def _store_zeros():
        out_ref[...] = jnp.zeros_like(logits)

    @pl.when(i == nsteps - 1)
    def _store_final():
        s = acc_ref[0]
        col = jax.lax.broadcasted_iota(jnp.int32, logits.shape, 1)
        a1 = jnp.min(jnp.where(is_max, col, _E), axis=1, keepdims=True)
        masked = jnp.where(col == a1, -jnp.inf, logits)
        a2 = jnp.min(jnp.where(masked == m2, col, _E), axis=1, keepdims=True)
        row = jax.lax.broadcasted_iota(jnp.int32, logits.shape, 0)
        vals = jnp.where(col == a1, m1 / s,
                         jnp.where(col == a2, m2 / s, 0.0))
        out_ref[...] = jnp.where(row == 0, vals, 0.0)


def kernel(x, W1, b1, W2, b2, W3, b3, W4, b4):
    b1r, b2r, b3r, b4r = (b.reshape(1, -1) for b in (b1, b2, b3, b4))

    full = lambda shape: pl.BlockSpec(shape, lambda i: (0, 0))
    return pl.pallas_call(
        _gate_kernel,
        grid=(_NBLK,),
        in_specs=[
            pl.BlockSpec((_BLK, _D), lambda i: (_NBLK - 1 - i, 0)),
            full((128, _D)), full((1, 128)),
            full((256, 128)), full((1, 256)),
            full((128, 256)), full((1, 128)),
            full((_E, 128)), full((1, _E)),
        ],
        out_specs=pl.BlockSpec((_BLK, _E), lambda i: (_NBLK - 1 - i, 0)),
        out_shape=jax.ShapeDtypeStruct((_B, _E), jnp.float32),
        scratch_shapes=[pltpu.SMEM((1,), jnp.float32)],
    )(x, W1, b1r, W2, b2r, W3, b3r, W4, b4r)


# speedup vs baseline: 1.0769x; 1.0769x over previous
import jax
import jax.numpy as jnp
from jax.experimental import pallas as pl
from jax.experimental.pallas import tpu as pltpu

_B, _D, _E = 8192, 4096, 64
_BLK = 1024
_NBLK = _B // _BLK

_NT = (((1,), (1,)), ((), ()))


def _leaky(h):
    return jnp.where(h >= 0, h, 0.01 * h)


def _ntdot(a, w):
    return jax.lax.dot_general(a, w, _NT, preferred_element_type=jnp.float32)


def _gate_kernel(xa_ref, xb_ref, w1_ref, b1_ref, w2_ref, b2_ref, w3_ref, b3_ref,
                 w4_ref, b4_ref, out_ref, acc_ref):
    i = pl.program_id(0)
    nsteps = pl.num_programs(0)

    @pl.when(i == 0)
    def _init():
        acc_ref[0] = 0.0

    h = _ntdot(xa_ref[...], w1_ref[:, :_D // 2])
    h += _ntdot(xb_ref[...], w1_ref[:, _D // 2:])
    h = jnp.maximum(h + b1_ref[...], 0.0)
    h = _leaky(_ntdot(h, w2_ref[...]) + b2_ref[...])
    h = _leaky(_ntdot(h, w3_ref[...]) + b3_ref[...])
    logits = _ntdot(h, w4_ref[...]) + b4_ref[...]

    m1 = jnp.max(logits, axis=1, keepdims=True)
    is_max = logits == m1
    dup = jnp.sum(is_max.astype(jnp.float32), axis=1, keepdims=True) > 1.0
    m2_lo = jnp.max(jnp.where(is_max, -jnp.inf, logits), axis=1, keepdims=True)
    m2 = jnp.where(dup, m1, m2_lo)
    acc_ref[0] += jnp.sum(m1) + jnp.sum(m2)

    @pl.when(i < nsteps - 1)
    def _store_zeros():
        out_ref[...] = jnp.zeros_like(logits)

    @pl.when(i == nsteps - 1)
    def _store_final():
        s = acc_ref[0]
        col = jax.lax.broadcasted_iota(jnp.int32, logits.shape, 1)
        a1 = jnp.min(jnp.where(is_max, col, _E), axis=1, keepdims=True)
        masked = jnp.where(col == a1, -jnp.inf, logits)
        a2 = jnp.min(jnp.where(masked == m2, col, _E), axis=1, keepdims=True)
        row = jax.lax.broadcasted_iota(jnp.int32, logits.shape, 0)
        vals = jnp.where(col == a1, m1 / s,
                         jnp.where(col == a2, m2 / s, 0.0))
        out_ref[...] = jnp.where(row == 0, vals, 0.0)


def kernel(x, W1, b1, W2, b2, W3, b3, W4, b4):
    b1r, b2r, b3r, b4r = (b.reshape(1, -1) for b in (b1, b2, b3, b4))

    full = lambda shape: pl.BlockSpec(shape, lambda i: (0, 0))
    return pl.pallas_call(
        _gate_kernel,
        grid=(_NBLK,),
        in_specs=[
            pl.BlockSpec((_BLK, _D // 2), lambda i: (_NBLK - 1 - i, 0)),
            pl.BlockSpec((_BLK, _D // 2), lambda i: (_NBLK - 1 - i, 1)),
            full((128, _D)), full((1, 128)),
            full((256, 128)), full((1, 256)),
            full((128, 256)), full((1, 128)),
            full((_E, 128)), full((1, _E)),
        ],
        out_specs=pl.BlockSpec((_BLK, _E), lambda i: (_NBLK - 1 - i, 0)),
        out_shape=jax.ShapeDtypeStruct((_B, _E), jnp.float32),
        scratch_shapes=[pltpu.SMEM((1,), jnp.float32)],
    )(x, x, W1, b1r, W2, b2r, W3, b3r, W4, b4r)
